# baseline (speedup 1.0000x reference)
import jax
import jax.numpy as jnp
from jax.experimental import pallas as pl
from jax.experimental.pallas import tpu as pltpu

_BLOCK_N = 4000


def _gnmax_body(x_ref, w_ref, o_ref):
    i = pl.program_id(0)
    h = jnp.sum(x_ref[...] * w_ref[...], axis=1)
    m = jnp.max(h)

    @pl.when(i == 0)
    def _init():
        o_ref[0] = m

    @pl.when(i > 0)
    def _acc():
        o_ref[0] = jnp.maximum(o_ref[0], m)


def kernel(x, W, b):
    n, d = x.shape
    grid = n // _BLOCK_N
    m = pl.pallas_call(
        _gnmax_body,
        grid=(grid,),
        in_specs=[
            pl.BlockSpec((_BLOCK_N, d), lambda i: (i, 0)),
            pl.BlockSpec((1, d), lambda i: (0, 0)),
        ],
        out_specs=pl.BlockSpec(memory_space=pltpu.SMEM),
        out_shape=jax.ShapeDtypeStruct((1,), jnp.float32),
    )(x, W)
    return m + b


# speedup vs baseline: 1.1476x; 1.1476x over previous
import jax
import jax.numpy as jnp
from jax.experimental import pallas as pl
from jax.experimental.pallas import tpu as pltpu

_BLOCK_N = 10000


def _gnmax_body(x_ref, w_ref, o_ref):
    i = pl.program_id(0)
    h = jnp.sum(x_ref[...] * w_ref[...], axis=1)
    m = jnp.max(h)

    @pl.when(i == 0)
    def _init():
        o_ref[0] = m

    @pl.when(i > 0)
    def _acc():
        o_ref[0] = jnp.maximum(o_ref[0], m)


def kernel(x, W, b):
    n, d = x.shape
    grid = n // _BLOCK_N
    m = pl.pallas_call(
        _gnmax_body,
        grid=(grid,),
        in_specs=[
            pl.BlockSpec((_BLOCK_N, d), lambda i: (i, 0)),
            pl.BlockSpec((1, d), lambda i: (0, 0)),
        ],
        out_specs=pl.BlockSpec(memory_space=pltpu.SMEM),
        out_shape=jax.ShapeDtypeStruct((1,), jnp.float32),
    )(x, W)
    return m + b
